# baseline (speedup 1.0000x reference)
import functools
import math

import jax
import jax.numpy as jnp
from jax import lax
from jax.experimental import pallas as pl
from jax.experimental.pallas import tpu as pltpu
from jax.experimental.pallas import tpu_sc as plsc

EPS = 1e-5
NC, NS, L = 2, 16, 16
NW = NC * NS
BLK = 128
EB = 32
AW = 128


def _prep_body(feat_ref, gamma_ref, beta_ref, wq_ref, bq_ref, wk_ref, wv_ref,
               qv_ref, kn_ref):
    x = feat_ref[...]
    n = x.shape[0]
    mean = jnp.sum(x, axis=0, keepdims=True) * (1.0 / n)
    xc = x - mean
    var = jnp.sum(xc * xc, axis=0, keepdims=True) * (1.0 / n)
    xn = xc * lax.rsqrt(var + EPS) * gamma_ref[...] + beta_ref[...]
    h = wq_ref.shape[1]
    qv_ref[:, :h] = -(jnp.dot(xn, wq_ref[...],
                              preferred_element_type=jnp.float32)
                      + bq_ref[...])
    qv_ref[:, h:] = jnp.dot(xn, wv_ref[...], preferred_element_type=jnp.float32)
    kn_ref[...] = -jnp.dot(xn, wk_ref[...], preferred_element_type=jnp.float32)


def _prep(feat, gamma, beta, wq, bq, wk, wv):
    n, d = feat.shape
    h = wq.shape[1]
    o = wv.shape[1]
    return pl.pallas_call(
        _prep_body,
        out_shape=[
            jax.ShapeDtypeStruct((n, h + o), jnp.float32),
            jax.ShapeDtypeStruct((n, h), jnp.float32),
        ],
    )(feat, gamma.reshape(1, d), beta.reshape(1, d), wq, bq.reshape(1, h),
      wk, wv)


def _make_edge_kernel(np_rows, np_den, nb, n_edges, ew):
    rows_per_tile = np_rows // NS
    chk = next(d for d in range(128, 0, -1) if rows_per_tile % d == 0)
    zc = next(d for d in range(min(EB, rows_per_tile), 0, -1)
              if rows_per_tile % d == 0)
    den_per_tile = np_den // NS
    mesh = plsc.VectorSubcoreMesh(core_axis_name="c", subcore_axis_name="s")

    @functools.partial(
        pl.kernel,
        out_type=[
            pltpu.HBM((NC, np_rows, AW), jnp.float32),
            pltpu.HBM((NC, np_den), jnp.float32),
        ],
        mesh=mesh,
        compiler_params=pltpu.CompilerParams(needs_layout_passes=False,
                                             use_tc_tiling_on_sc=False),
        scratch_types=[
            pltpu.VMEM((nb, EB // 2), jnp.int32),
            pltpu.VMEM((nb, EB), jnp.int32),
            [pltpu.VMEM((EB,), jnp.int32)] * 2,
            [pltpu.VMEM((EB, 256), jnp.float32)] * 2,
            [pltpu.VMEM((EB, 128), jnp.float32)] * 2,
            [pltpu.VMEM((EB, 128), jnp.float32)] * 2,
            [pltpu.VMEM((EB,), jnp.float32)] * 2,
            pltpu.VMEM((128,), jnp.float32),
            pltpu.VMEM((BLK,), jnp.float32),
            pltpu.VMEM_SHARED((np_rows, AW), jnp.float32),
            pltpu.VMEM_SHARED((np_den,), jnp.float32),
            [pltpu.SemaphoreType.DMA] * 2,
            [pltpu.SemaphoreType.DMA] * 2,
        ],
    )
    def edge_kernel(qv_hbm, kn_hbm, src_hbm, dst_hbm, we_hbm,
                    out_hbm, den_hbm,
                    src_pk, dst_v, srcs, qvbufs, kbufs, obufs, wbufs,
                    we_v, zblk, acc, dacc, gsems, ssems):
        c = lax.axis_index("c")
        s = lax.axis_index("s")
        wid = c * NS + s

        zero16 = jnp.zeros((L,), jnp.float32)

        @pl.loop(0, EB)
        def _zero_rows(r):
            for i in range(AW // L):
                obufs[0][r, i * L:(i + 1) * L] = zero16

        @pl.loop(0, BLK // L)
        def _zero_w(r):
            zblk[pl.ds(r * L, L)] = zero16

        base = s * rows_per_tile

        @pl.loop(0, rows_per_tile // zc)
        def _zero_acc(b):
            pltpu.sync_copy(obufs[0].at[pl.ds(0, zc)],
                            acc.at[pl.ds(base + b * zc, zc)])

        dbase = s * den_per_tile

        @pl.loop(0, den_per_tile // BLK)
        def _zero_dacc(b):
            pltpu.sync_copy(zblk, dacc.at[pl.ds(dbase + b * BLK, BLK)])

        pltpu.sync_copy(src_hbm.at[wid], src_pk)
        pltpu.sync_copy(dst_hbm.at[wid], dst_v)
        pltpu.sync_copy(we_hbm, we_v)
        wec = [we_v[i * L:(i + 1) * L] for i in range(128 // L)]
        lane_ids = lax.iota(jnp.int32, L)
        lane0_mask = lane_ids == 0
        edge_base = wid * ew

        plsc.subcore_barrier()

        def unpack_src(b, jj):
            for h in range(EB // 32):
                pk = src_pk[jj, pl.ds(h * L, L)]
                srcs[b][pl.ds(h * 32, L)] = pk & 0xFFFF
                srcs[b][pl.ds(h * 32 + L, L)] = lax.shift_right_logical(
                    pk, 16)

        def start_gathers(b, jj):
            pltpu.async_copy(qv_hbm.at[srcs[b]], qvbufs[b], gsems[b])
            pltpu.async_copy(kn_hbm.at[dst_v.at[jj]], kbufs[b], gsems[b])

        def wait_gathers(b, jj):
            pltpu.make_async_copy(qv_hbm.at[srcs[b]], qvbufs[b],
                                  gsems[b]).wait()
            pltpu.make_async_copy(kn_hbm.at[dst_v.at[jj]], kbufs[b],
                                  gsems[b]).wait()

        def start_scatters(b, jj):
            pltpu.async_copy(obufs[b], acc.at[dst_v.at[jj]], ssems[b],
                             add=True)
            pltpu.async_copy(wbufs[b], dacc.at[dst_v.at[jj]], ssems[b],
                             add=True)

        def wait_scatters(b, jj):
            pltpu.make_async_copy(obufs[b], acc.at[dst_v.at[jj]],
                                  ssems[b]).wait()
            pltpu.make_async_copy(wbufs[b], dacc.at[dst_v.at[jj]],
                                  ssems[b]).wait()

        for b in range(2):
            unpack_src(b, b)
            start_gathers(b, b)

        @pl.loop(0, nb, step=2)
        def _pair(j):
            for b in range(2):
                jj = j + b
                wait_gathers(b, jj)

                @pl.when(jj >= 2)
                def _drain():
                    wait_scatters(b, jj - 2)

                qvbuf, kbuf, obuf, wbuf = (
                    qvbufs[b], kbufs[b], obufs[b], wbufs[b])
                blk_base = edge_base + jj * EB

                @plsc.parallel_loop(0, EB, unroll=8)
                def _edge(e):
                    accv = zero16
                    for i in range(128 // L):
                        t = jnp.exp(qvbuf[e, i * L:(i + 1) * L]
                                    + kbuf[e, i * L:(i + 1) * L])
                        accv = accv + wec[i] / (1.0 + t)
                    w = jnp.exp(jnp.broadcast_to(jnp.sum(accv), (L,)))
                    w = jnp.where(blk_base + e < n_edges, w, 0.0)
                    for i in range(128 // L):
                        obuf[e, i * L:(i + 1) * L] = (
                            w * qvbuf[e, 128 + i * L:128 + (i + 1) * L])
                    plsc.store_scatter(wbuf, [jnp.full((L,), e, jnp.int32)],
                                       w, mask=lane0_mask)

                start_scatters(b, jj)

                @pl.when(jj + 2 < nb)
                def _prefetch():
                    unpack_src(b, jj + 2)
                    start_gathers(b, jj + 2)

        wait_scatters(0, nb - 2)
        wait_scatters(1, nb - 1)

        plsc.subcore_barrier()

        @pl.loop(0, rows_per_tile // chk)
        def _writeback(b):
            off = base + b * chk
            pltpu.sync_copy(acc.at[pl.ds(off, chk)],
                            out_hbm.at[c, pl.ds(off, chk)])

        @pl.loop(0, den_per_tile // BLK)
        def _writeback_den(b):
            off = dbase + b * BLK
            pltpu.sync_copy(dacc.at[pl.ds(off, BLK)],
                            den_hbm.at[c, pl.ds(off, BLK)])

    return edge_kernel


def _fin_body(n, o, acc_ref, den_ref, out_ref):
    num = acc_ref[0, :n, :o] + acc_ref[1, :n, :o]
    den = (den_ref[0, :n] + den_ref[1, :n])[:, None]
    out_ref[...] = jnp.where(den != 0.0, num / den, 0.0)


def _finalize(acc, den, n, o):
    return pl.pallas_call(
        functools.partial(_fin_body, n, o),
        out_shape=jax.ShapeDtypeStruct((n, o), jnp.float32),
    )(acc, den)


def kernel(feat, edge_index, gamma, beta, Wq, bq, Wk, Wv, We):
    n, d = feat.shape
    o = Wv.shape[1]
    e = edge_index.shape[1]

    qv, kn = _prep(feat, gamma, beta, Wq, bq, Wk, Wv)

    ew = math.ceil(e / (NW * 2 * EB)) * (2 * EB)
    ep = ew * NW
    nb = ew // EB
    src = jnp.concatenate(
        [edge_index[0], jnp.zeros((ep - e,), jnp.int32)]).reshape(NW, nb, EB)
    dst = jnp.concatenate(
        [edge_index[1], jnp.zeros((ep - e,), jnp.int32)]).reshape(NW, nb, EB)
    s3 = src.reshape(NW, nb, EB // 32, 2, 16)
    src_pk = (s3[:, :, :, 0, :] | (s3[:, :, :, 1, :] << 16)).reshape(
        NW, nb, EB // 2)

    np_rows = math.ceil(n / NS) * NS
    np_den = math.ceil(n / (NS * BLK)) * (NS * BLK)
    edge_kernel = _make_edge_kernel(np_rows, np_den, nb, e, ew)
    acc, den = edge_kernel(qv, kn, src_pk, dst, We.reshape(-1))

    return _finalize(acc, den, n, o)


# speedup vs baseline: 1.6400x; 1.6400x over previous
import functools
import math

import jax
import jax.numpy as jnp
from jax import lax
from jax.experimental import pallas as pl
from jax.experimental.pallas import tpu as pltpu
from jax.experimental.pallas import tpu_sc as plsc

EPS = 1e-5
NC, NS, L = 2, 16, 16
NW = NC * NS
BLK = 128
EB = 32
AW = 128


def _prep_body(feat_ref, gamma_ref, beta_ref, wq_ref, bq_ref, wk_ref, wv_ref,
               qv_ref, kn_ref):
    x = feat_ref[...]
    n = x.shape[0]
    mean = jnp.sum(x, axis=0, keepdims=True) * (1.0 / n)
    xc = x - mean
    var = jnp.sum(xc * xc, axis=0, keepdims=True) * (1.0 / n)
    xn = xc * lax.rsqrt(var + EPS) * gamma_ref[...] + beta_ref[...]
    h = wq_ref.shape[1]
    qv_ref[:, :h] = -(jnp.dot(xn, wq_ref[...],
                              preferred_element_type=jnp.float32)
                      + bq_ref[...])
    qv_ref[:, h:] = jnp.dot(xn, wv_ref[...], preferred_element_type=jnp.float32)
    kn_ref[...] = -jnp.dot(xn, wk_ref[...], preferred_element_type=jnp.float32)


def _prep(feat, gamma, beta, wq, bq, wk, wv):
    n, d = feat.shape
    h = wq.shape[1]
    o = wv.shape[1]
    return pl.pallas_call(
        _prep_body,
        out_shape=[
            jax.ShapeDtypeStruct((n, h + o), jnp.float32),
            jax.ShapeDtypeStruct((n, h), jnp.float32),
        ],
    )(feat, gamma.reshape(1, d), beta.reshape(1, d), wq, bq.reshape(1, h),
      wk, wv)


def _make_edge_kernel(np_rows, np_den, nb, n_edges, ew):
    rows_per_tile = np_rows // NS
    chk = next(d for d in range(128, 0, -1) if rows_per_tile % d == 0)
    zc = next(d for d in range(min(EB, rows_per_tile), 0, -1)
              if rows_per_tile % d == 0)
    den_per_tile = np_den // NS
    mesh = plsc.VectorSubcoreMesh(core_axis_name="c", subcore_axis_name="s")

    @functools.partial(
        pl.kernel,
        out_type=[
            pltpu.HBM((NC, np_rows, AW), jnp.float32),
            pltpu.HBM((NC, np_den), jnp.float32),
        ],
        mesh=mesh,
        compiler_params=pltpu.CompilerParams(needs_layout_passes=False,
                                             use_tc_tiling_on_sc=False),
        scratch_types=[
            pltpu.VMEM((nb, EB // 2), jnp.int32),
            pltpu.VMEM((nb, EB), jnp.int32),
            [pltpu.VMEM((EB,), jnp.int32)] * 2,
            [pltpu.VMEM((EB, 256), jnp.bfloat16)] * 2,
            [pltpu.VMEM((EB, 128), jnp.bfloat16)] * 2,
            [pltpu.VMEM((EB, 128), jnp.float32)] * 2,
            [pltpu.VMEM((EB,), jnp.float32)] * 2,
            pltpu.VMEM((128,), jnp.float32),
            pltpu.VMEM((BLK,), jnp.float32),
            pltpu.VMEM_SHARED((np_rows, AW), jnp.float32),
            pltpu.VMEM_SHARED((np_den,), jnp.float32),
            [pltpu.SemaphoreType.DMA] * 2,
            [pltpu.SemaphoreType.DMA] * 2,
        ],
    )
    def edge_kernel(qv_hbm, kn_hbm, src_hbm, dst_hbm, we_hbm,
                    out_hbm, den_hbm,
                    src_pk, dst_v, srcs, qvbufs, kbufs, obufs, wbufs,
                    we_v, zblk, acc, dacc, gsems, ssems):
        c = lax.axis_index("c")
        s = lax.axis_index("s")
        wid = c * NS + s

        zero16 = jnp.zeros((L,), jnp.float32)

        @pl.loop(0, EB)
        def _zero_rows(r):
            for i in range(AW // L):
                obufs[0][r, i * L:(i + 1) * L] = zero16

        @pl.loop(0, BLK // L)
        def _zero_w(r):
            zblk[pl.ds(r * L, L)] = zero16

        base = s * rows_per_tile

        @pl.loop(0, rows_per_tile // zc)
        def _zero_acc(b):
            pltpu.sync_copy(obufs[0].at[pl.ds(0, zc)],
                            acc.at[pl.ds(base + b * zc, zc)])

        dbase = s * den_per_tile

        @pl.loop(0, den_per_tile // BLK)
        def _zero_dacc(b):
            pltpu.sync_copy(zblk, dacc.at[pl.ds(dbase + b * BLK, BLK)])

        pltpu.sync_copy(src_hbm.at[wid], src_pk)
        pltpu.sync_copy(dst_hbm.at[wid], dst_v)
        pltpu.sync_copy(we_hbm, we_v)
        wec = [we_v[i * L:(i + 1) * L] for i in range(128 // L)]
        lane_ids = lax.iota(jnp.int32, L)
        lane0_mask = lane_ids == 0
        edge_base = wid * ew

        plsc.subcore_barrier()

        def unpack_src(b, jj):
            for h in range(EB // 32):
                pk = src_pk[jj, pl.ds(h * L, L)]
                srcs[b][pl.ds(h * 32, L)] = pk & 0xFFFF
                srcs[b][pl.ds(h * 32 + L, L)] = lax.shift_right_logical(
                    pk, 16)

        def start_gathers(b, jj):
            pltpu.async_copy(qv_hbm.at[srcs[b]], qvbufs[b], gsems[b])
            pltpu.async_copy(kn_hbm.at[dst_v.at[jj]], kbufs[b], gsems[b])

        def wait_gathers(b, jj):
            pltpu.make_async_copy(qv_hbm.at[srcs[b]], qvbufs[b],
                                  gsems[b]).wait()
            pltpu.make_async_copy(kn_hbm.at[dst_v.at[jj]], kbufs[b],
                                  gsems[b]).wait()

        def start_scatters(b, jj):
            pltpu.async_copy(obufs[b], acc.at[dst_v.at[jj]], ssems[b],
                             add=True)
            pltpu.async_copy(wbufs[b], dacc.at[dst_v.at[jj]], ssems[b],
                             add=True)

        def wait_scatters(b, jj):
            pltpu.make_async_copy(obufs[b], acc.at[dst_v.at[jj]],
                                  ssems[b]).wait()
            pltpu.make_async_copy(wbufs[b], dacc.at[dst_v.at[jj]],
                                  ssems[b]).wait()

        for b in range(2):
            unpack_src(b, b)
            start_gathers(b, b)

        @pl.loop(0, nb, step=2)
        def _pair(j):
            for b in range(2):
                jj = j + b
                wait_gathers(b, jj)

                @pl.when(jj >= 2)
                def _drain():
                    wait_scatters(b, jj - 2)

                qvbuf, kbuf, obuf, wbuf = (
                    qvbufs[b], kbufs[b], obufs[b], wbufs[b])
                blk_base = edge_base + jj * EB

                @plsc.parallel_loop(0, EB, unroll=4)
                def _edge(e):
                    accv = zero16
                    for i in range(128 // 32):
                        za, zb = plsc.unpack(
                            qvbuf[e, 32 * i:32 * (i + 1)],
                            format=plsc.PackFormat.INTERLEAVED,
                            preferred_element_type=jnp.float32)
                        ka, kb = plsc.unpack(
                            kbuf[e, 32 * i:32 * (i + 1)],
                            format=plsc.PackFormat.INTERLEAVED,
                            preferred_element_type=jnp.float32)
                        accv = accv + wec[2 * i] / (1.0 + jnp.exp(za + ka))
                        accv = (accv
                                + wec[2 * i + 1] / (1.0 + jnp.exp(zb + kb)))
                    w = jnp.exp(jnp.broadcast_to(jnp.sum(accv), (L,)))
                    w = jnp.where(blk_base + e < n_edges, w, 0.0)
                    for i in range(128 // 32):
                        va, vb = plsc.unpack(
                            qvbuf[e, 128 + 32 * i:128 + 32 * (i + 1)],
                            format=plsc.PackFormat.INTERLEAVED,
                            preferred_element_type=jnp.float32)
                        obuf[e, 32 * i:32 * i + L] = w * va
                        obuf[e, 32 * i + L:32 * (i + 1)] = w * vb
                    plsc.store_scatter(wbuf, [jnp.full((L,), e, jnp.int32)],
                                       w, mask=lane0_mask)

                start_scatters(b, jj)

                @pl.when(jj + 2 < nb)
                def _prefetch():
                    unpack_src(b, jj + 2)
                    start_gathers(b, jj + 2)

        wait_scatters(0, nb - 2)
        wait_scatters(1, nb - 1)

        plsc.subcore_barrier()

        @pl.loop(0, rows_per_tile // chk)
        def _writeback(b):
            off = base + b * chk
            pltpu.sync_copy(acc.at[pl.ds(off, chk)],
                            out_hbm.at[c, pl.ds(off, chk)])

        @pl.loop(0, den_per_tile // BLK)
        def _writeback_den(b):
            off = dbase + b * BLK
            pltpu.sync_copy(dacc.at[pl.ds(off, BLK)],
                            den_hbm.at[c, pl.ds(off, BLK)])

    return edge_kernel


def _fin_body(n, o, acc_ref, den_ref, out_ref):
    num = acc_ref[0, :n, :o] + acc_ref[1, :n, :o]
    den = (den_ref[0, :n] + den_ref[1, :n])[:, None]
    out_ref[...] = jnp.where(den != 0.0, num / den, 0.0)


def _finalize(acc, den, n, o):
    return pl.pallas_call(
        functools.partial(_fin_body, n, o),
        out_shape=jax.ShapeDtypeStruct((n, o), jnp.float32),
    )(acc, den)


def _ileave(x):
    n0, f = x.shape
    y = x.reshape(n0, f // 32, 2, 16)
    return jnp.transpose(y, (0, 1, 3, 2)).reshape(n0, f)


def kernel(feat, edge_index, gamma, beta, Wq, bq, Wk, Wv, We):
    n, d = feat.shape
    o = Wv.shape[1]
    e = edge_index.shape[1]

    qv, kn = _prep(feat, gamma, beta, Wq, bq, Wk, Wv)
    qv = _ileave(qv).astype(jnp.bfloat16)
    kn = _ileave(kn).astype(jnp.bfloat16)

    ew = math.ceil(e / (NW * 2 * EB)) * (2 * EB)
    ep = ew * NW
    nb = ew // EB
    src = jnp.concatenate(
        [edge_index[0], jnp.zeros((ep - e,), jnp.int32)]).reshape(NW, nb, EB)
    dst = jnp.concatenate(
        [edge_index[1], jnp.zeros((ep - e,), jnp.int32)]).reshape(NW, nb, EB)
    s3 = src.reshape(NW, nb, EB // 32, 2, 16)
    src_pk = (s3[:, :, :, 0, :] | (s3[:, :, :, 1, :] << 16)).reshape(
        NW, nb, EB // 2)

    np_rows = math.ceil(n / NS) * NS
    np_den = math.ceil(n / (NS * BLK)) * (NS * BLK)
    edge_kernel = _make_edge_kernel(np_rows, np_den, nb, e, ew)
    acc, den = edge_kernel(qv, kn, src_pk, dst, We.reshape(-1))

    return _finalize(acc, den, n, o)


# speedup vs baseline: 1.7947x; 1.0943x over previous
import functools
import math

import jax
import jax.numpy as jnp
from jax import lax
from jax.experimental import pallas as pl
from jax.experimental.pallas import tpu as pltpu
from jax.experimental.pallas import tpu_sc as plsc

EPS = 1e-5
NC, NS, L = 2, 16, 16
NW = NC * NS
BLK = 128
EB = 48
AW = 128


def _prep_body(feat_ref, gamma_ref, beta_ref, wq_ref, bq_ref, wk_ref, wv_ref,
               qv_ref, kn_ref):
    x = feat_ref[...]
    n = x.shape[0]
    mean = jnp.sum(x, axis=0, keepdims=True) * (1.0 / n)
    xc = x - mean
    var = jnp.sum(xc * xc, axis=0, keepdims=True) * (1.0 / n)
    xn = xc * lax.rsqrt(var + EPS) * gamma_ref[...] + beta_ref[...]
    h = wq_ref.shape[1]
    qv_ref[:, :h] = -(jnp.dot(xn, wq_ref[...],
                              preferred_element_type=jnp.float32)
                      + bq_ref[...])
    qv_ref[:, h:] = jnp.dot(xn, wv_ref[...], preferred_element_type=jnp.float32)
    kn_ref[...] = -jnp.dot(xn, wk_ref[...], preferred_element_type=jnp.float32)


def _prep(feat, gamma, beta, wq, bq, wk, wv):
    n, d = feat.shape
    h = wq.shape[1]
    o = wv.shape[1]
    return pl.pallas_call(
        _prep_body,
        out_shape=[
            jax.ShapeDtypeStruct((n, h + o), jnp.float32),
            jax.ShapeDtypeStruct((n, h), jnp.float32),
        ],
    )(feat, gamma.reshape(1, d), beta.reshape(1, d), wq, bq.reshape(1, h),
      wk, wv)


def _make_edge_kernel(np_rows, np_den, nb, n_edges, ew):
    rows_per_tile = np_rows // NS
    chk = next(d for d in range(128, 0, -1) if rows_per_tile % d == 0)
    zc = next(d for d in range(min(EB, rows_per_tile), 0, -1)
              if rows_per_tile % d == 0)
    den_per_tile = np_den // NS
    mesh = plsc.VectorSubcoreMesh(core_axis_name="c", subcore_axis_name="s")

    @functools.partial(
        pl.kernel,
        out_type=[
            pltpu.HBM((NC, np_rows, AW), jnp.float32),
            pltpu.HBM((NC, np_den), jnp.float32),
        ],
        mesh=mesh,
        compiler_params=pltpu.CompilerParams(needs_layout_passes=False,
                                             use_tc_tiling_on_sc=False),
        scratch_types=[
            pltpu.VMEM((nb, 32), jnp.int32),
            pltpu.VMEM((nb, EB), jnp.int32),
            [pltpu.VMEM((EB,), jnp.int32)] * 2,
            [pltpu.VMEM((EB, 256), jnp.bfloat16)] * 2,
            [pltpu.VMEM((EB, 128), jnp.bfloat16)] * 2,
            [pltpu.VMEM((EB, 128), jnp.float32)] * 2,
            [pltpu.VMEM((EB,), jnp.float32)] * 2,
            pltpu.VMEM((128,), jnp.float32),
            pltpu.VMEM((BLK,), jnp.float32),
            pltpu.VMEM_SHARED((np_rows, AW), jnp.float32),
            pltpu.VMEM_SHARED((np_den,), jnp.float32),
            [pltpu.SemaphoreType.DMA] * 2,
            [pltpu.SemaphoreType.DMA] * 2,
        ],
    )
    def edge_kernel(qv_hbm, kn_hbm, src_hbm, dst_hbm, we_hbm,
                    out_hbm, den_hbm,
                    src_c, dst_v, srcs, qvbufs, kbufs, obufs, wbufs,
                    we_v, zblk, acc, dacc, gsems, ssems):
        c = lax.axis_index("c")
        s = lax.axis_index("s")
        wid = c * NS + s

        zero16 = jnp.zeros((L,), jnp.float32)

        @pl.loop(0, EB)
        def _zero_rows(r):
            for i in range(AW // L):
                obufs[0][r, i * L:(i + 1) * L] = zero16

        @pl.loop(0, BLK // L)
        def _zero_w(r):
            zblk[pl.ds(r * L, L)] = zero16

        base = s * rows_per_tile

        @pl.loop(0, rows_per_tile // zc)
        def _zero_acc(b):
            pltpu.sync_copy(obufs[0].at[pl.ds(0, zc)],
                            acc.at[pl.ds(base + b * zc, zc)])

        dbase = s * den_per_tile

        @pl.loop(0, den_per_tile // BLK)
        def _zero_dacc(b):
            pltpu.sync_copy(zblk, dacc.at[pl.ds(dbase + b * BLK, BLK)])

        pltpu.sync_copy(src_hbm.at[wid], src_c)
        pltpu.sync_copy(dst_hbm.at[wid], dst_v)
        pltpu.sync_copy(we_hbm, we_v)
        wec = [we_v[i * L:(i + 1) * L] for i in range(128 // L)]
        lane_ids = lax.iota(jnp.int32, L)
        lane0_mask = lane_ids == 0
        edge_base = wid * ew

        plsc.subcore_barrier()

        def unpack_src(b, jj):
            pk = src_c[jj, 0:L]
            srcs[b][pl.ds(0, L)] = pk & 0xFFFF
            srcs[b][pl.ds(L, L)] = lax.shift_right_logical(pk, 16)
            srcs[b][pl.ds(2 * L, L)] = src_c[jj, L:2 * L]

        def start_gathers(b, jj):
            pltpu.async_copy(qv_hbm.at[srcs[b]], qvbufs[b], gsems[b])
            pltpu.async_copy(kn_hbm.at[dst_v.at[jj]], kbufs[b], gsems[b])

        def wait_gathers(b, jj):
            pltpu.make_async_copy(qv_hbm.at[srcs[b]], qvbufs[b],
                                  gsems[b]).wait()
            pltpu.make_async_copy(kn_hbm.at[dst_v.at[jj]], kbufs[b],
                                  gsems[b]).wait()

        def start_scatters(b, jj):
            pltpu.async_copy(obufs[b], acc.at[dst_v.at[jj]], ssems[b],
                             add=True)
            pltpu.async_copy(wbufs[b], dacc.at[dst_v.at[jj]], ssems[b],
                             add=True)

        def wait_scatters(b, jj):
            pltpu.make_async_copy(obufs[b], acc.at[dst_v.at[jj]],
                                  ssems[b]).wait()
            pltpu.make_async_copy(wbufs[b], dacc.at[dst_v.at[jj]],
                                  ssems[b]).wait()

        for b in range(2):
            unpack_src(b, b)
            start_gathers(b, b)

        @pl.loop(0, nb, step=2)
        def _pair(j):
            for b in range(2):
                jj = j + b
                wait_gathers(b, jj)

                @pl.when(jj >= 2)
                def _drain():
                    wait_scatters(b, jj - 2)

                qvbuf, kbuf, obuf, wbuf = (
                    qvbufs[b], kbufs[b], obufs[b], wbufs[b])
                blk_base = edge_base + jj * EB

                @plsc.parallel_loop(0, EB, unroll=4)
                def _edge(e):
                    accv = zero16
                    for i in range(128 // 32):
                        za, zb = plsc.unpack(
                            qvbuf[e, 32 * i:32 * (i + 1)],
                            format=plsc.PackFormat.INTERLEAVED,
                            preferred_element_type=jnp.float32)
                        ka, kb = plsc.unpack(
                            kbuf[e, 32 * i:32 * (i + 1)],
                            format=plsc.PackFormat.INTERLEAVED,
                            preferred_element_type=jnp.float32)
                        accv = accv + wec[2 * i] / (1.0 + jnp.exp(za + ka))
                        accv = (accv
                                + wec[2 * i + 1] / (1.0 + jnp.exp(zb + kb)))
                    w = jnp.exp(jnp.broadcast_to(jnp.sum(accv), (L,)))
                    w = jnp.where(blk_base + e < n_edges, w, 0.0)
                    for i in range(128 // 32):
                        va, vb = plsc.unpack(
                            qvbuf[e, 128 + 32 * i:128 + 32 * (i + 1)],
                            format=plsc.PackFormat.INTERLEAVED,
                            preferred_element_type=jnp.float32)
                        obuf[e, 32 * i:32 * i + L] = w * va
                        obuf[e, 32 * i + L:32 * (i + 1)] = w * vb
                    plsc.store_scatter(wbuf, [jnp.full((L,), e, jnp.int32)],
                                       w, mask=lane0_mask)

                start_scatters(b, jj)

                @pl.when(jj + 2 < nb)
                def _prefetch():
                    unpack_src(b, jj + 2)
                    start_gathers(b, jj + 2)

        wait_scatters(0, nb - 2)
        wait_scatters(1, nb - 1)

        plsc.subcore_barrier()

        @pl.loop(0, rows_per_tile // chk)
        def _writeback(b):
            off = base + b * chk
            pltpu.sync_copy(acc.at[pl.ds(off, chk)],
                            out_hbm.at[c, pl.ds(off, chk)])

        @pl.loop(0, den_per_tile // BLK)
        def _writeback_den(b):
            off = dbase + b * BLK
            pltpu.sync_copy(dacc.at[pl.ds(off, BLK)],
                            den_hbm.at[c, pl.ds(off, BLK)])

    return edge_kernel


def _fin_body(n, o, acc_ref, den_ref, out_ref):
    num = acc_ref[0, :n, :o] + acc_ref[1, :n, :o]
    den = (den_ref[0, :n] + den_ref[1, :n])[:, None]
    out_ref[...] = jnp.where(den != 0.0, num / den, 0.0)


def _finalize(acc, den, n, o):
    return pl.pallas_call(
        functools.partial(_fin_body, n, o),
        out_shape=jax.ShapeDtypeStruct((n, o), jnp.float32),
    )(acc, den)


def _ileave(x):
    n0, f = x.shape
    y = x.reshape(n0, f // 32, 2, 16)
    return jnp.transpose(y, (0, 1, 3, 2)).reshape(n0, f)


def kernel(feat, edge_index, gamma, beta, Wq, bq, Wk, Wv, We):
    n, d = feat.shape
    o = Wv.shape[1]
    e = edge_index.shape[1]

    qv, kn = _prep(feat, gamma, beta, Wq, bq, Wk, Wv)
    qv = _ileave(qv).astype(jnp.bfloat16)
    kn = _ileave(kn).astype(jnp.bfloat16)

    ew = math.ceil(e / (NW * 2 * EB)) * (2 * EB)
    ep = ew * NW
    nb = ew // EB
    src = jnp.concatenate(
        [edge_index[0], jnp.zeros((ep - e,), jnp.int32)]).reshape(NW, nb, EB)
    src_c = jnp.concatenate(
        [src[:, :, 0:16] | (src[:, :, 16:32] << 16), src[:, :, 32:48]],
        axis=2)
    dst = jnp.concatenate(
        [edge_index[1], jnp.zeros((ep - e,), jnp.int32)]).reshape(NW, nb, EB)

    np_rows = math.ceil(n / NS) * NS
    np_den = math.ceil(n / (NS * BLK)) * (NS * BLK)
    edge_kernel = _make_edge_kernel(np_rows, np_den, nb, e, ew)
    acc, den = edge_kernel(qv, kn, src_c, dst, We.reshape(-1))

    return _finalize(acc, den, n, o)
